# baseline (speedup 1.0000x reference)
import jax
import jax.numpy as jnp
from jax import lax
from jax.experimental import pallas as pl
from jax.experimental.pallas import tpu as pltpu
from jax.experimental.pallas import tpu_sc as plsc

N = 50000
D = 128
H = 128
C = 16
E = 160000
R = 3

NPAD = 51200
SUBN = NPAD // 16
NF = 50176
NCHUNK = 4
CH = NF // NCHUNK
SUBC = CH // 16
EPAD = 163840
ECH = EPAD // 128

_MESH = plsc.VectorSubcoreMesh(core_axis_name="c", subcore_axis_name="s")


def _zero_vmem_2d(buf, rows):
    z = jnp.zeros((16,), jnp.float32)

    def body(i, _):
        r = i // 8
        q = i % 8
        buf[r, pl.ds(q * 16, 16)] = z
        return 0

    lax.fori_loop(0, rows * 8, body, 0)


def _zero_vmem_1d(buf, n):
    z = jnp.zeros((16,), jnp.float32)

    def body(i, _):
        buf[pl.ds(i * 16, 16)] = z
        return 0

    lax.fori_loop(0, n // 16, body, 0)


def _hist_kernel(eflat, out, sbuf, dbuf, ones, zb, shs, shd):
    c = lax.axis_index("c")
    s = lax.axis_index("s")
    t = s * 2 + c

    _zero_vmem_1d(zb, SUBN)
    ob = jnp.ones((16,), jnp.float32)

    def ones_body(i, _):
        ones[i // 8, pl.ds((i % 8) * 16, 16)] = ob
        return 0

    lax.fori_loop(0, 64, ones_body, 0)

    for r in range(R):
        pltpu.sync_copy(zb, shs.at[pl.ds(s * SUBN, SUBN)])
        pltpu.sync_copy(zb, shd.at[pl.ds(s * SUBN, SUBN)])
        plsc.subcore_barrier()

        base_src = (r * 2 + 0) * ECH
        base_dst = (r * 2 + 1) * ECH

        for blk in range(5):
            row0 = t * 40 + blk * 8
            pltpu.sync_copy(eflat.at[pl.ds(base_src + row0, 8)], sbuf)
            pltpu.sync_copy(eflat.at[pl.ds(base_dst + row0, 8)], dbuf)
            for k in range(8):
                pltpu.sync_copy(ones.at[k], shs.at[sbuf.at[k]], add=True)
                pltpu.sync_copy(ones.at[k], shd.at[dbuf.at[k]], add=True)

        plsc.subcore_barrier()
        pltpu.sync_copy(shs.at[pl.ds(s * SUBN, SUBN)],
                        out.at[pl.ds((r * 4 + c) * NPAD + s * SUBN, SUBN)])
        pltpu.sync_copy(shd.at[pl.ds(s * SUBN, SUBN)],
                        out.at[pl.ds((r * 4 + 2 + c) * NPAD + s * SUBN, SUBN)])
        plsc.subcore_barrier()


def _hist_call(eflat):
    k = pl.kernel(
        _hist_kernel,
        out_type=jax.ShapeDtypeStruct((12 * NPAD,), jnp.float32),
        mesh=_MESH,
        scratch_types=[
            pltpu.VMEM((8, 128), jnp.int32),
            pltpu.VMEM((8, 128), jnp.int32),
            pltpu.VMEM((8, 128), jnp.float32),
            pltpu.VMEM((SUBN,), jnp.float32),
            pltpu.VMEM_SHARED((NPAD,), jnp.float32),
            pltpu.VMEM_SHARED((NPAD,), jnp.float32),
        ],
    )
    return k(eflat)


def _prescale_body(f_ref, hp_ref, h_ref, ns_ref, nd_ref):
    f = f_ref[...]
    for r in range(R):
        outd = hp_ref[r * 4 + 0, :] + hp_ref[r * 4 + 1, :]
        ind = hp_ref[r * 4 + 2, :] + hp_ref[r * 4 + 3, :]
        ns = lax.rsqrt(jnp.maximum(outd, 1.0))
        nd = lax.rsqrt(jnp.maximum(ind, 1.0))
        h_ref[r, :, :] = f * ns[:, None]
        ns_ref[r, :] = ns
        nd_ref[r, :] = nd


def _prescale_call(featp, hpart):
    B = 512
    grid = NF // B
    return pl.pallas_call(
        _prescale_body,
        grid=(grid,),
        in_specs=[
            pl.BlockSpec((B, 128), lambda i: (i, 0)),
            pl.BlockSpec((12, B), lambda i: (0, i)),
        ],
        out_specs=[
            pl.BlockSpec((R, B, 128), lambda i: (0, i, 0)),
            pl.BlockSpec((R, B), lambda i: (0, i)),
            pl.BlockSpec((R, B), lambda i: (0, i)),
        ],
        out_shape=[
            jax.ShapeDtypeStruct((R, NF, 128), jnp.float32),
            jax.ShapeDtypeStruct((R, NF), jnp.float32),
            jax.ShapeDtypeStruct((R, NF), jnp.float32),
        ],
    )(featp, hpart)


def _cpart_kernel(eflat, ndst, out, sbuf, dbuf, valb, zb, ct, sem):
    c = lax.axis_index("c")
    s = lax.axis_index("s")
    t = s * 2 + c

    _zero_vmem_1d(zb, SUBN)

    for r in range(R):
        pltpu.sync_copy(zb, ct.at[pl.ds(s * SUBN, SUBN)])
        plsc.subcore_barrier()

        base_src = (r * 2 + 0) * ECH
        base_dst = (r * 2 + 1) * ECH
        off = jnp.full((16,), r * NF, jnp.int32)

        for blk in range(5):
            row0 = t * 40 + blk * 8
            pltpu.sync_copy(eflat.at[pl.ds(base_src + row0, 8)], sbuf)
            pltpu.sync_copy(eflat.at[pl.ds(base_dst + row0, 8)], dbuf)
            if r > 0:
                def adj(i, _):
                    k = i // 8
                    q = i % 8
                    dbuf[k, pl.ds(q * 16, 16)] = \
                        dbuf[k, pl.ds(q * 16, 16)] + off
                    return 0
                lax.fori_loop(0, 64, adj, 0)
            for k in range(8):
                pltpu.async_copy(ndst.at[dbuf.at[k]], valb.at[k], sem).wait()
            for k in range(8):
                pltpu.sync_copy(valb.at[k], ct.at[sbuf.at[k]], add=True)

        plsc.subcore_barrier()
        pltpu.sync_copy(ct.at[pl.ds(s * SUBN, SUBN)],
                        out.at[pl.ds((r * 2 + c) * NPAD + s * SUBN, SUBN)])
        plsc.subcore_barrier()


def _cpart_call(eflat, ndst):
    k = pl.kernel(
        _cpart_kernel,
        out_type=jax.ShapeDtypeStruct((6 * NPAD,), jnp.float32),
        mesh=_MESH,
        scratch_types=[
            pltpu.VMEM((8, 128), jnp.int32),
            pltpu.VMEM((8, 128), jnp.int32),
            pltpu.VMEM((8, 128), jnp.float32),
            pltpu.VMEM((SUBN,), jnp.float32),
            pltpu.VMEM_SHARED((NPAD,), jnp.float32),
            pltpu.SemaphoreType.DMA,
        ],
    )
    return k(eflat, ndst)


def _agg_kernel(eflat, hflat, aggflat,
                sbuf, dbuf, gidx, ridx, rows, zbuf, sem, chunk):
    c = lax.axis_index("c")
    s = lax.axis_index("s")

    _zero_vmem_2d(zbuf, 16)

    for r in range(R):
        base_src = (r * 2 + 0) * ECH
        base_dst = (r * 2 + 1) * ECH
        for p in range(2):
            kchunk = p * 2 + c
            lo = kchunk * CH

            for m in range(49):
                pltpu.sync_copy(zbuf, chunk.at[pl.ds(s * SUBC + m * 16, 16)])
            plsc.subcore_barrier()

            def blk_body(blk, _):
                row0 = s * 80 + blk * 8
                pltpu.sync_copy(eflat.at[pl.ds(base_src + row0, 8)], sbuf)
                pltpu.sync_copy(eflat.at[pl.ds(base_dst + row0, 8)], dbuf)

                def vbody(i, _):
                    k = i // 8
                    q = i % 8
                    sv = sbuf[k, pl.ds(q * 16, 16)]
                    dv = dbuf[k, pl.ds(q * 16, 16)]
                    inm = (dv >= lo) & (dv < lo + CH)
                    gidx[k, pl.ds(q * 16, 16)] = \
                        jnp.where(inm, sv + r * NF, r * NF + N)
                    ridx[k, pl.ds(q * 16, 16)] = \
                        jnp.clip(dv - lo, 0, CH - 1)
                    return 0

                lax.fori_loop(0, 64, vbody, 0)

                for k in range(8):
                    pltpu.async_copy(hflat.at[gidx.at[k]], rows, sem).wait()
                    pltpu.sync_copy(rows, chunk.at[ridx.at[k]], add=True)
                return 0

            lax.fori_loop(0, 10, blk_body, 0)

            plsc.subcore_barrier()
            pltpu.sync_copy(
                chunk.at[pl.ds(s * SUBC, SUBC)],
                aggflat.at[pl.ds(r * NF + lo + s * SUBC, SUBC)])
            plsc.subcore_barrier()


def _agg_call(eflat, hflat):
    k = pl.kernel(
        _agg_kernel,
        out_type=jax.ShapeDtypeStruct((R * NF, 128), jnp.float32),
        mesh=_MESH,
        scratch_types=[
            pltpu.VMEM((8, 128), jnp.int32),
            pltpu.VMEM((8, 128), jnp.int32),
            pltpu.VMEM((8, 128), jnp.int32),
            pltpu.VMEM((8, 128), jnp.int32),
            pltpu.VMEM((128, 128), jnp.float32),
            pltpu.VMEM((16, 128), jnp.float32),
            pltpu.SemaphoreType.DMA,
            pltpu.VMEM_SHARED((CH, 128), jnp.float32),
        ],
    )
    return k(eflat, hflat)


def _final_body(agg_ref, nd_ref, ns_ref, cp_ref, w1_ref, b1_ref,
                w2_ref, b2_ref, wc_ref, bc_ref, out_ref, acc):
    i = pl.program_id(0)

    b1sum = b1_ref[0, :] + b1_ref[1, :] + b1_ref[2, :]
    x = jnp.zeros((512, 128), jnp.float32)
    for r in range(R):
        scaled = agg_ref[r] * nd_ref[r, :][:, None]
        x = x + lax.dot(scaled, w1_ref[r],
                        precision=lax.Precision.HIGHEST,
                        preferred_element_type=jnp.float32)
    h1 = jnp.maximum(x + b1sum[None, :], 0.0)

    rowid = i * 512 + lax.broadcasted_iota(jnp.int32, (512, 1), 0)
    valid = (rowid < N).astype(jnp.float32)

    @pl.when(i == 0)
    def _():
        acc[...] = jnp.zeros((R, 128), jnp.float32)

    for r in range(R):
        cw = ns_ref[r, :] * (cp_ref[r * 2, :] + cp_ref[r * 2 + 1, :])
        acc[r, :] += jnp.sum(h1 * (cw[:, None] * valid), axis=0)

    @pl.when(i == pl.num_programs(0) - 1)
    def _():
        hg = jnp.zeros((1, 128), jnp.float32)
        for r in range(R):
            hg = hg + lax.dot(acc[r, :][None, :], w2_ref[r],
                              precision=lax.Precision.HIGHEST,
                              preferred_element_type=jnp.float32)
            hg = hg + float(N) * b2_ref[r, :][None, :]
        out_ref[...] = lax.dot(hg, wc_ref[...],
                               precision=lax.Precision.HIGHEST,
                               preferred_element_type=jnp.float32) \
            + bc_ref[...]


def _final_call(aggflat, ndst, nsrc, cpart, W1, b1, W2, b2, Wc, bc):
    B = 512
    grid = NF // B
    agg3 = aggflat.reshape(R, NF, 128)
    return pl.pallas_call(
        _final_body,
        grid=(grid,),
        in_specs=[
            pl.BlockSpec((R, B, 128), lambda i: (0, i, 0)),
            pl.BlockSpec((R, B), lambda i: (0, i)),
            pl.BlockSpec((R, B), lambda i: (0, i)),
            pl.BlockSpec((6, B), lambda i: (0, i)),
            pl.BlockSpec((R, 128, 128), lambda i: (0, 0, 0)),
            pl.BlockSpec((R, 128), lambda i: (0, 0)),
            pl.BlockSpec((R, 128, 128), lambda i: (0, 0, 0)),
            pl.BlockSpec((R, 128), lambda i: (0, 0)),
            pl.BlockSpec((128, C), lambda i: (0, 0)),
            pl.BlockSpec((1, C), lambda i: (0, 0)),
        ],
        out_specs=pl.BlockSpec((1, C), lambda i: (0, 0)),
        out_shape=jax.ShapeDtypeStruct((1, C), jnp.float32),
        scratch_shapes=[pltpu.VMEM((R, 128), jnp.float32)],
    )(agg3, ndst, nsrc, cpart, W1, b1, W2, b2, Wc, bc.reshape(1, C))


def kernel(feat, edge_index_follows, edge_index_likes, edge_index_owns,
           W1, b1, W2, b2, Wc, bc):
    epad = [jnp.pad(e, ((0, 0), (0, EPAD - E)), constant_values=N)
            for e in (edge_index_follows, edge_index_likes, edge_index_owns)]
    eflat = jnp.stack(epad).reshape(R * 2 * ECH, 128)
    featp = jnp.pad(feat, ((0, NF - N), (0, 0)))

    hpart = _hist_call(eflat).reshape(12, NPAD)
    h_all, nsrc, ndst = _prescale_call(featp, hpart)
    cpart = _cpart_call(eflat, ndst.reshape(R * NF)).reshape(6, NPAD)
    hflat = h_all.reshape(R * NF, 128)
    aggflat = _agg_call(eflat, hflat)
    return _final_call(aggflat, ndst, nsrc, cpart,
                       W1, b1, W2, b2, Wc, bc)


# speedup vs baseline: 1.0004x; 1.0004x over previous
import jax
import jax.numpy as jnp
from jax import lax
from jax.experimental import pallas as pl
from jax.experimental.pallas import tpu as pltpu
from jax.experimental.pallas import tpu_sc as plsc

N = 50000
D = 128
H = 128
C = 16
E = 160000
R = 3

NPAD = 51200
SUBN = NPAD // 16
NF = 50176
NCHUNK = 4
CH = NF // NCHUNK
SUBC = CH // 16
EPAD = 163840
ECH = EPAD // 128

_MESH = plsc.VectorSubcoreMesh(core_axis_name="c", subcore_axis_name="s")


def _zero_vmem_2d(buf, rows):
    z = jnp.zeros((16,), jnp.float32)

    def body(i, _):
        r = i // 8
        q = i % 8
        buf[r, pl.ds(q * 16, 16)] = z
        return 0

    lax.fori_loop(0, rows * 8, body, 0)


def _zero_vmem_1d(buf, n):
    z = jnp.zeros((16,), jnp.float32)

    def body(i, _):
        buf[pl.ds(i * 16, 16)] = z
        return 0

    lax.fori_loop(0, n // 16, body, 0)


def _hist_kernel(eflat, out, sbuf, dbuf, ones, zb, shs, shd):
    c = lax.axis_index("c")
    s = lax.axis_index("s")
    t = s * 2 + c

    _zero_vmem_1d(zb, SUBN)
    ob = jnp.ones((16,), jnp.float32)

    def ones_body(i, _):
        ones[i // 8, pl.ds((i % 8) * 16, 16)] = ob
        return 0

    lax.fori_loop(0, 64, ones_body, 0)

    for r in range(R):
        pltpu.sync_copy(zb, shs.at[pl.ds(s * SUBN, SUBN)])
        pltpu.sync_copy(zb, shd.at[pl.ds(s * SUBN, SUBN)])
        plsc.subcore_barrier()

        base_src = (r * 2 + 0) * ECH
        base_dst = (r * 2 + 1) * ECH

        for blk in range(5):
            row0 = t * 40 + blk * 8
            pltpu.sync_copy(eflat.at[pl.ds(base_src + row0, 8)], sbuf)
            pltpu.sync_copy(eflat.at[pl.ds(base_dst + row0, 8)], dbuf)
            for k in range(8):
                pltpu.sync_copy(ones.at[k], shs.at[sbuf.at[k]], add=True)
                pltpu.sync_copy(ones.at[k], shd.at[dbuf.at[k]], add=True)

        plsc.subcore_barrier()
        pltpu.sync_copy(shs.at[pl.ds(s * SUBN, SUBN)],
                        out.at[pl.ds((r * 4 + c) * NPAD + s * SUBN, SUBN)])
        pltpu.sync_copy(shd.at[pl.ds(s * SUBN, SUBN)],
                        out.at[pl.ds((r * 4 + 2 + c) * NPAD + s * SUBN, SUBN)])
        plsc.subcore_barrier()


def _hist_call(eflat):
    k = pl.kernel(
        _hist_kernel,
        out_type=jax.ShapeDtypeStruct((12 * NPAD,), jnp.float32),
        mesh=_MESH,
        scratch_types=[
            pltpu.VMEM((8, 128), jnp.int32),
            pltpu.VMEM((8, 128), jnp.int32),
            pltpu.VMEM((8, 128), jnp.float32),
            pltpu.VMEM((SUBN,), jnp.float32),
            pltpu.VMEM_SHARED((NPAD,), jnp.float32),
            pltpu.VMEM_SHARED((NPAD,), jnp.float32),
        ],
    )
    return k(eflat)


def _prescale_body(f_ref, hp_ref, h_ref, ns_ref, nd_ref):
    f = f_ref[...]
    for r in range(R):
        outd = hp_ref[r * 4 + 0, :] + hp_ref[r * 4 + 1, :]
        ind = hp_ref[r * 4 + 2, :] + hp_ref[r * 4 + 3, :]
        ns = lax.rsqrt(jnp.maximum(outd, 1.0))
        nd = lax.rsqrt(jnp.maximum(ind, 1.0))
        h_ref[r, :, :] = f * ns[:, None]
        ns_ref[r, :] = ns
        nd_ref[r, :] = nd


def _prescale_call(featp, hpart):
    B = 512
    grid = NF // B
    return pl.pallas_call(
        _prescale_body,
        grid=(grid,),
        in_specs=[
            pl.BlockSpec((B, 128), lambda i: (i, 0)),
            pl.BlockSpec((12, B), lambda i: (0, i)),
        ],
        out_specs=[
            pl.BlockSpec((R, B, 128), lambda i: (0, i, 0)),
            pl.BlockSpec((R, B), lambda i: (0, i)),
            pl.BlockSpec((R, B), lambda i: (0, i)),
        ],
        out_shape=[
            jax.ShapeDtypeStruct((R, NF, 128), jnp.float32),
            jax.ShapeDtypeStruct((R, NF), jnp.float32),
            jax.ShapeDtypeStruct((R, NF), jnp.float32),
        ],
    )(featp, hpart)


def _cpart_kernel(eflat, ndst, out, sbuf, dbuf, valb, zb, ct, sem):
    c = lax.axis_index("c")
    s = lax.axis_index("s")
    t = s * 2 + c

    _zero_vmem_1d(zb, SUBN)

    for r in range(R):
        pltpu.sync_copy(zb, ct.at[pl.ds(s * SUBN, SUBN)])
        plsc.subcore_barrier()

        base_src = (r * 2 + 0) * ECH
        base_dst = (r * 2 + 1) * ECH
        off = jnp.full((16,), r * NF, jnp.int32)

        for blk in range(5):
            row0 = t * 40 + blk * 8
            pltpu.sync_copy(eflat.at[pl.ds(base_src + row0, 8)], sbuf)
            pltpu.sync_copy(eflat.at[pl.ds(base_dst + row0, 8)], dbuf)
            if r > 0:
                def adj(i, _):
                    k = i // 8
                    q = i % 8
                    dbuf[k, pl.ds(q * 16, 16)] = \
                        dbuf[k, pl.ds(q * 16, 16)] + off
                    return 0
                lax.fori_loop(0, 64, adj, 0)
            for k in range(8):
                pltpu.async_copy(ndst.at[dbuf.at[k]], valb.at[k], sem).wait()
            for k in range(8):
                pltpu.sync_copy(valb.at[k], ct.at[sbuf.at[k]], add=True)

        plsc.subcore_barrier()
        pltpu.sync_copy(ct.at[pl.ds(s * SUBN, SUBN)],
                        out.at[pl.ds((r * 2 + c) * NPAD + s * SUBN, SUBN)])
        plsc.subcore_barrier()


def _cpart_call(eflat, ndst):
    k = pl.kernel(
        _cpart_kernel,
        out_type=jax.ShapeDtypeStruct((6 * NPAD,), jnp.float32),
        mesh=_MESH,
        scratch_types=[
            pltpu.VMEM((8, 128), jnp.int32),
            pltpu.VMEM((8, 128), jnp.int32),
            pltpu.VMEM((8, 128), jnp.float32),
            pltpu.VMEM((SUBN,), jnp.float32),
            pltpu.VMEM_SHARED((NPAD,), jnp.float32),
            pltpu.SemaphoreType.DMA,
        ],
    )
    return k(eflat, ndst)


def _agg_kernel(eflat, hflat, aggflat,
                sbuf, dbuf, gidx, ridx, rows, zbuf, sem, chunk):
    c = lax.axis_index("c")
    s = lax.axis_index("s")

    _zero_vmem_2d(zbuf, 16)

    for r in range(R):
        base_src = (r * 2 + 0) * ECH
        base_dst = (r * 2 + 1) * ECH
        for p in range(2):
            kchunk = p * 2 + c
            lo = kchunk * CH

            for m in range(49):
                pltpu.sync_copy(zbuf, chunk.at[pl.ds(s * SUBC + m * 16, 16)])
            plsc.subcore_barrier()

            def blk_body(blk, _):
                row0 = s * 80 + blk * 8
                pltpu.sync_copy(eflat.at[pl.ds(base_src + row0, 8)], sbuf)
                pltpu.sync_copy(eflat.at[pl.ds(base_dst + row0, 8)], dbuf)

                def vbody(i, _):
                    k = i // 8
                    q = i % 8
                    sv = sbuf[k, pl.ds(q * 16, 16)]
                    dv = dbuf[k, pl.ds(q * 16, 16)]
                    inm = (dv >= lo) & (dv < lo + CH)
                    gidx[k, pl.ds(q * 16, 16)] = \
                        jnp.where(inm, sv + r * NF, r * NF + N)
                    ridx[k, pl.ds(q * 16, 16)] = \
                        jnp.clip(dv - lo, 0, CH - 1)
                    return 0

                lax.fori_loop(0, 64, vbody, 0)

                for k in range(8):
                    pltpu.async_copy(hflat.at[gidx.at[k]], rows, sem).wait()
                    pltpu.sync_copy(rows, chunk.at[ridx.at[k]], add=True)
                return 0

            lax.fori_loop(0, 10, blk_body, 0)

            plsc.subcore_barrier()
            pltpu.sync_copy(
                chunk.at[pl.ds(s * SUBC, SUBC)],
                aggflat.at[pl.ds(r * NF + lo + s * SUBC, SUBC)])
            plsc.subcore_barrier()


def _agg_call(eflat, hflat):
    k = pl.kernel(
        _agg_kernel,
        out_type=jax.ShapeDtypeStruct((R * NF, 128), jnp.float32),
        mesh=_MESH,
        scratch_types=[
            pltpu.VMEM((8, 128), jnp.int32),
            pltpu.VMEM((8, 128), jnp.int32),
            pltpu.VMEM((8, 128), jnp.int32),
            pltpu.VMEM((8, 128), jnp.int32),
            pltpu.VMEM((128, 128), jnp.float32),
            pltpu.VMEM((16, 128), jnp.float32),
            pltpu.SemaphoreType.DMA,
            pltpu.VMEM_SHARED((CH, 128), jnp.float32),
        ],
    )
    return k(eflat, hflat)


def _final_body(agg_ref, nd_ref, ns_ref, cp_ref, w1_ref, b1_ref,
                w2_ref, b2_ref, wc_ref, bc_ref, out_ref, acc):
    i = pl.program_id(0)

    rowid = i * 512 + lax.broadcasted_iota(jnp.int32, (512, 1), 0)
    valid = (rowid < N).astype(jnp.float32)

    b1sum = b1_ref[0, :] + b1_ref[1, :] + b1_ref[2, :]
    x = jnp.zeros((512, 128), jnp.float32)
    for r in range(R):
        scaled = agg_ref[r] * nd_ref[r, :][:, None]
        x = x + lax.dot(scaled, w1_ref[r],
                        precision=lax.Precision.HIGHEST,
                        preferred_element_type=jnp.float32)
    h1 = jnp.maximum(x + b1sum[None, :], 0.0)

    @pl.when(i == 0)
    def _():
        acc[...] = jnp.zeros((R, 128), jnp.float32)

    for r in range(R):
        cw = ns_ref[r, :] * (cp_ref[r * 2, :] + cp_ref[r * 2 + 1, :])
        acc[r, :] += jnp.sum(h1 * (cw[:, None] * valid), axis=0)

    @pl.when(i == pl.num_programs(0) - 1)
    def _():
        hg = jnp.zeros((1, 128), jnp.float32)
        for r in range(R):
            hg = hg + lax.dot(acc[r, :][None, :], w2_ref[r],
                              precision=lax.Precision.HIGHEST,
                              preferred_element_type=jnp.float32)
            hg = hg + float(N) * b2_ref[r, :][None, :]
        out_ref[...] = lax.dot(hg, wc_ref[...],
                               precision=lax.Precision.HIGHEST,
                               preferred_element_type=jnp.float32) \
            + bc_ref[...]


def _final_call(aggflat, ndst, nsrc, cpart, W1, b1, W2, b2, Wc, bc):
    B = 512
    grid = NF // B
    agg3 = aggflat.reshape(R, NF, 128)
    return pl.pallas_call(
        _final_body,
        grid=(grid,),
        in_specs=[
            pl.BlockSpec((R, B, 128), lambda i: (0, i, 0)),
            pl.BlockSpec((R, B), lambda i: (0, i)),
            pl.BlockSpec((R, B), lambda i: (0, i)),
            pl.BlockSpec((6, B), lambda i: (0, i)),
            pl.BlockSpec((R, 128, 128), lambda i: (0, 0, 0)),
            pl.BlockSpec((R, 128), lambda i: (0, 0)),
            pl.BlockSpec((R, 128, 128), lambda i: (0, 0, 0)),
            pl.BlockSpec((R, 128), lambda i: (0, 0)),
            pl.BlockSpec((128, C), lambda i: (0, 0)),
            pl.BlockSpec((1, C), lambda i: (0, 0)),
        ],
        out_specs=pl.BlockSpec((1, C), lambda i: (0, 0)),
        out_shape=jax.ShapeDtypeStruct((1, C), jnp.float32),
        scratch_shapes=[pltpu.VMEM((R, 128), jnp.float32)],
    )(agg3, ndst, nsrc, cpart, W1, b1, W2, b2, Wc, bc.reshape(1, C))


def kernel(feat, edge_index_follows, edge_index_likes, edge_index_owns,
           W1, b1, W2, b2, Wc, bc):
    epad = [jnp.pad(e, ((0, 0), (0, EPAD - E)), constant_values=N)
            for e in (edge_index_follows, edge_index_likes, edge_index_owns)]
    eflat = jnp.stack(epad).reshape(R * 2 * ECH, 128)
    featp = jnp.pad(feat, ((0, NF - N), (0, 0)))

    hpart = _hist_call(eflat).reshape(12, NPAD)
    h_all, nsrc, ndst = _prescale_call(featp, hpart)
    cpart = _cpart_call(eflat, ndst.reshape(R * NF)).reshape(6, NPAD)
    hflat = h_all.reshape(R * NF, 128)
    aggflat = _agg_call(eflat, hflat)
    return _final_call(aggflat, ndst, nsrc, cpart,
                       W1, b1, W2, b2, Wc, bc)
